# baseline (speedup 1.0000x reference)
import functools

import jax
import jax.numpy as jnp
from jax import lax
from jax.experimental import pallas as pl
from jax.experimental.pallas import tpu as pltpu
from jax.experimental.pallas import tpu_sc as plsc

NC = 2
NS = 16
NW = NC * NS
CHUNK = 128
LANES = 16


def _sc_aggregate(x, src_slab, dst_slab, n_pad, nchunk):
    D = x.shape[1]
    rows_per_tile = n_pad // NS
    n_init = rows_per_tile // CHUNK
    mesh = plsc.VectorSubcoreMesh(
        core_axis_name="c", subcore_axis_name="s",
        num_cores=NC, num_subcores=NS)

    @functools.partial(
        pl.kernel,
        out_type=jax.ShapeDtypeStruct((NC, n_pad, D), jnp.float32),
        mesh=mesh,
        scratch_types=[
            pltpu.VMEM((CHUNK,), jnp.int32),
            pltpu.VMEM((CHUNK,), jnp.int32),
            pltpu.VMEM((CHUNK,), jnp.int32),
            pltpu.VMEM((CHUNK,), jnp.int32),
            pltpu.VMEM((CHUNK,), jnp.int32),
            pltpu.VMEM((CHUNK,), jnp.int32),
            pltpu.VMEM((CHUNK,), jnp.int32),
            pltpu.VMEM((CHUNK,), jnp.int32),
            pltpu.VMEM((CHUNK, D), jnp.float32),
            pltpu.VMEM((CHUNK, D), jnp.float32),
            pltpu.VMEM_SHARED((n_pad, D), jnp.float32),
            pltpu.SemaphoreType.DMA,
            pltpu.SemaphoreType.DMA,
            pltpu.SemaphoreType.DMA,
        ],
    )
    def agg(x_hbm, src_hbm, dst_hbm, out_hbm,
            sidx0, sidx1, sidx2, sidx3, didx0, didx1, didx2, didx3,
            rows0, rows1, acc_sh, sem_g, sem_si, sem_di):
        c = lax.axis_index("c")
        s = lax.axis_index("s")
        wid = s * NC + c
        row0 = s * rows_per_tile
        sidx = (sidx0, sidx1, sidx2, sidx3)
        didx = (didx0, didx1, didx2, didx3)
        rows = (rows0, rows1)

        for b in range(4):
            pltpu.async_copy(src_hbm.at[wid, b], sidx[b], sem_si)
            pltpu.async_copy(dst_hbm.at[wid, b], didx[b], sem_di)

        def zrow(r, _):
            for cc in range(D // LANES):
                rows0[r, pl.ds(cc * LANES, LANES)] = jnp.zeros(
                    (LANES,), jnp.float32)
            return 0
        lax.fori_loop(0, CHUNK, zrow, 0)
        for t in range(n_init):
            pltpu.sync_copy(rows0,
                            acc_sh.at[pl.ds(row0 + t * CHUNK, CHUNK)])

        pltpu.make_async_copy(src_hbm.at[wid, 0], sidx0, sem_si).wait()
        pltpu.async_copy(x_hbm.at[sidx0], rows0, sem_g)
        pltpu.make_async_copy(src_hbm.at[wid, 0], sidx1, sem_si).wait()
        pltpu.async_copy(x_hbm.at[sidx1], rows1, sem_g)
        plsc.subcore_barrier()

        def body(i, _):
            j = 4 * i
            for p in range(4):
                jj = j + p
                buf = rows[p % 2]
                pltpu.make_async_copy(
                    x_hbm.at[sidx[p]], buf, sem_g).wait()
                pltpu.make_async_copy(
                    dst_hbm.at[wid, 0], didx[p], sem_di).wait()
                pltpu.sync_copy(buf, acc_sh.at[didx[p]], add=True)
                pltpu.make_async_copy(
                    src_hbm.at[wid, 0], sidx[(p + 2) % 4], sem_si).wait()
                pltpu.async_copy(x_hbm.at[sidx[(p + 2) % 4]], buf, sem_g)
                pltpu.async_copy(src_hbm.at[wid, jj + 4], sidx[p], sem_si)
                pltpu.async_copy(dst_hbm.at[wid, jj + 4], didx[p], sem_di)
            return 0
        lax.fori_loop(0, nchunk // 4, body, 0)
        pltpu.make_async_copy(x_hbm.at[sidx0], rows0, sem_g).wait()
        pltpu.make_async_copy(x_hbm.at[sidx1], rows1, sem_g).wait()
        for b in range(2):
            pltpu.make_async_copy(
                src_hbm.at[wid, 0], sidx[b], sem_si).wait()
        for b in range(4):
            pltpu.make_async_copy(
                dst_hbm.at[wid, 0], didx[b], sem_di).wait()

        plsc.subcore_barrier()
        pltpu.sync_copy(acc_sh.at[pl.ds(row0, rows_per_tile)],
                        out_hbm.at[c, pl.ds(row0, rows_per_tile)])

    return agg(x, src_slab, dst_slab)


def _mlp(x, p0, p1, W1, b1, W2, b2):
    N, D = x.shape
    BLK = 1024

    def body(x_ref, p0_ref, p1_ref, w1_ref, b1_ref, w2_ref, b2_ref, o_ref):
        h = x_ref[...] + p0_ref[...] + p1_ref[...]
        h = jnp.dot(h, w1_ref[...], preferred_element_type=jnp.float32)
        h = jnp.maximum(h + b1_ref[...], 0.0)
        o = jnp.dot(h, w2_ref[...], preferred_element_type=jnp.float32)
        o_ref[...] = o + b2_ref[...]

    grid = (pl.cdiv(N, BLK),)
    row_spec = pl.BlockSpec((BLK, D), lambda i: (i, 0))
    full = lambda shape: pl.BlockSpec(shape, lambda i: (0, 0))
    return pl.pallas_call(
        body,
        grid=grid,
        in_specs=[row_spec, row_spec, row_spec,
                  full((D, D)), full((1, D)), full((D, D)), full((1, D))],
        out_specs=row_spec,
        out_shape=jax.ShapeDtypeStruct((N, D), jnp.float32),
    )(x, p0, p1, W1, b1.reshape(1, D), W2, b2.reshape(1, D))


def kernel(x, edge_index, W1, b1, W2, b2):
    N, D = x.shape
    E = edge_index.shape[1]
    rows_per_tile = -(-N // (NS * CHUNK)) * CHUNK
    n_pad = rows_per_tile * NS

    e_per_w = -(-E // NW)
    nchunk = -(-e_per_w // CHUNK)
    nchunk = -(-nchunk // 4) * 4
    e_pad = nchunk * CHUNK

    src = edge_index[0]
    dst = edge_index[1]
    pad_n = NW * e_pad - E
    src_slab = jnp.pad(src, (0, pad_n)).reshape(NW, nchunk, CHUNK)
    dst_slab = jnp.pad(dst, (0, pad_n),
                       constant_values=N).reshape(NW, nchunk, CHUNK)
    dummy_src = jnp.zeros((NW, 4, CHUNK), jnp.int32)
    dummy_dst = jnp.full((NW, 4, CHUNK), N, jnp.int32)
    src_slab = jnp.concatenate([src_slab, dummy_src], axis=1)
    dst_slab = jnp.concatenate([dst_slab, dummy_dst], axis=1)

    p = _sc_aggregate(x, src_slab, dst_slab, n_pad, nchunk)
    out = _mlp(x, p[0, :N], p[1, :N], W1, b1, W2, b2)
    return out


# speedup vs baseline: 2.1750x; 2.1750x over previous
import functools

import jax
import jax.numpy as jnp
from jax import lax
from jax.experimental import pallas as pl
from jax.experimental.pallas import tpu as pltpu
from jax.experimental.pallas import tpu_sc as plsc

NC = 2
NS = 16
NW = NC * NS
CHUNK = 128
LANES = 16


def _sc_aggregate(x, src_slab, dst_slab, n_pad, nchunk):
    D = x.shape[1]
    rows_per_tile = n_pad // NS
    n_init = rows_per_tile // CHUNK
    mesh = plsc.VectorSubcoreMesh(
        core_axis_name="c", subcore_axis_name="s",
        num_cores=NC, num_subcores=NS)

    @functools.partial(
        pl.kernel,
        out_type=jax.ShapeDtypeStruct((NC, n_pad, D), jnp.float32),
        mesh=mesh,
        scratch_types=[
            pltpu.VMEM((nchunk, CHUNK), jnp.int32),
            pltpu.VMEM((nchunk, CHUNK), jnp.int32),
            pltpu.VMEM((CHUNK, D), jnp.float32),
            pltpu.VMEM_SHARED((n_pad, D), jnp.float32),
            pltpu.SemaphoreType.DMA,
        ],
    )
    def agg(x_hbm, src_hbm, dst_hbm, out_hbm, src_v, dst_v, rows_v, acc_sh, sem):
        c = lax.axis_index("c")
        s = lax.axis_index("s")
        wid = c * NS + s
        row0 = s * rows_per_tile

        def zrow(r, _):
            for cc in range(D // LANES):
                rows_v[r, pl.ds(cc * LANES, LANES)] = jnp.zeros(
                    (LANES,), jnp.float32)
            return 0
        lax.fori_loop(0, CHUNK, zrow, 0)
        for t in range(n_init):
            pltpu.sync_copy(rows_v,
                            acc_sh.at[pl.ds(row0 + t * CHUNK, CHUNK)])

        pltpu.sync_copy(src_hbm.at[wid], src_v)
        pltpu.sync_copy(dst_hbm.at[wid], dst_v)
        plsc.subcore_barrier()

        def body(j, _):
            pltpu.async_copy(x_hbm.at[src_v.at[j]], rows_v, sem).wait()
            pltpu.sync_copy(rows_v, acc_sh.at[dst_v.at[j]], add=True)
            return 0
        lax.fori_loop(0, nchunk, body, 0)

        plsc.subcore_barrier()
        pltpu.sync_copy(acc_sh.at[pl.ds(row0, rows_per_tile)],
                        out_hbm.at[c, pl.ds(row0, rows_per_tile)])

    return agg(x, src_slab, dst_slab)


def _mlp(x, p0, p1, W1, b1, W2, b2):
    N, D = x.shape
    BLK = 2048

    def body(x_ref, p0_ref, p1_ref, w1_ref, b1_ref, w2_ref, b2_ref, o_ref):
        h = x_ref[...] + p0_ref[...] + p1_ref[...]
        h = jnp.dot(h, w1_ref[...], preferred_element_type=jnp.float32)
        h = jnp.maximum(h + b1_ref[...], 0.0)
        o = jnp.dot(h, w2_ref[...], preferred_element_type=jnp.float32)
        o_ref[...] = o + b2_ref[...]

    grid = (pl.cdiv(N, BLK),)
    row_spec = pl.BlockSpec((BLK, D), lambda i: (i, 0))
    full = lambda shape: pl.BlockSpec(shape, lambda i: (0, 0))
    return pl.pallas_call(
        body,
        grid=grid,
        in_specs=[row_spec, row_spec, row_spec,
                  full((D, D)), full((1, D)), full((D, D)), full((1, D))],
        out_specs=row_spec,
        out_shape=jax.ShapeDtypeStruct((N, D), jnp.float32),
    )(x, p0, p1, W1, b1.reshape(1, D), W2, b2.reshape(1, D))


def kernel(x, edge_index, W1, b1, W2, b2):
    N, D = x.shape
    E = edge_index.shape[1]
    rows_per_tile = -(-N // (NS * CHUNK)) * CHUNK
    n_pad = rows_per_tile * NS

    e_per_w = -(-E // NW)
    nchunk = -(-e_per_w // CHUNK)
    e_pad = nchunk * CHUNK

    src = edge_index[0]
    dst = edge_index[1]
    pad_n = NW * e_pad - E
    src_slab = jnp.pad(src, (0, pad_n)).reshape(NW, nchunk, CHUNK)
    dst_slab = jnp.pad(dst, (0, pad_n),
                       constant_values=N).reshape(NW, nchunk, CHUNK)

    p = _sc_aggregate(x, src_slab, dst_slab, n_pad, nchunk)
    out = _mlp(x, p[0, :N], p[1, :N], W1, b1, W2, b2)
    return out
